# baseline (speedup 1.0000x reference)
import jax
import jax.numpy as jnp
from jax import lax
from jax.experimental import pallas as pl
from jax.experimental.pallas import tpu as pltpu
from jax.experimental.pallas import tpu_sc as plsc

_BN = 512
_S = 1024
_NW = 32
_RPW = _S // _NW
_T = 16
_R = 4
_K = 4096
_B = 8
_L = 16
_NCHUNK = _K // _L


def _sc_body(x_hbm, nw_hbm, ow_hbm, o1_hbm, o2_hbm, x_v, w_v, o_v):
    cid = lax.axis_index("c")
    sid = lax.axis_index("s")
    wid = sid * 2 + cid
    col0 = wid * _RPW
    pltpu.sync_copy(x_hbm, x_v)
    lanes = lax.iota(jnp.int32, _L)
    for w_hbm, o_hbm in ((nw_hbm, o1_hbm), (ow_hbm, o2_hbm)):
        for t in range(_RPW // _T):
            pltpu.sync_copy(w_hbm.at[pl.ds(col0 + t * _T, _T), :], w_v)
            res = [jnp.zeros((_L,), jnp.float32) for _ in range(_B)]
            for g in range(_T // _R):
                def chunk(i, accs):
                    b0 = i * _L
                    xs = [x_v[b, pl.ds(b0, _L)] for b in range(_B)]
                    ws = [w_v[g * _R + r, pl.ds(b0, _L)] for r in range(_R)]
                    return tuple(accs[b * _R + r] + xs[b] * ws[r]
                                 for b in range(_B) for r in range(_R))
                init = tuple(jnp.zeros((_L,), jnp.float32)
                             for _ in range(_B * _R))
                accs = lax.fori_loop(0, _NCHUNK, chunk, init)
                for b in range(_B):
                    for r in range(_R):
                        v = accs[b * _R + r]
                        for sh in (8, 4, 2, 1):
                            v = v + v.at[lanes ^ sh].get(
                                mode="promise_in_bounds")
                        res[b] = jnp.where(lanes == g * _R + r, v, res[b])
            for b in range(_B):
                o_v[pl.ds(b * _RPW + t * _T, _L)] = res[b]
        pltpu.sync_copy(o_v, o_hbm.at[wid])


def _tc_body(x_ref, nw_ref, ow_ref, o1_ref, o2_ref):
    x = x_ref[...]
    dims = (((1,), (1,)), ((), ()))
    o1_ref[...] = jax.lax.dot_general(
        x, nw_ref[...], dims, preferred_element_type=jnp.float32)
    o2_ref[...] = jax.lax.dot_general(
        x, ow_ref[...], dims, preferred_element_type=jnp.float32)


def _make_sc_call():
    mesh = plsc.VectorSubcoreMesh(
        core_axis_name="c", subcore_axis_name="s",
        num_cores=2, num_subcores=16)
    return pl.kernel(
        _sc_body,
        out_type=[jax.ShapeDtypeStruct((_NW, _B * _RPW), jnp.float32)] * 2,
        mesh=mesh,
        scratch_types=[
            pltpu.VMEM((_B, _K), jnp.float32),
            pltpu.VMEM((_T, _K), jnp.float32),
            pltpu.VMEM((_B * _RPW,), jnp.float32),
        ],
    )


@jax.jit
def kernel(x, new_weight, orig_weight):
    b, k = x.shape
    n = new_weight.shape[0]
    sc1, sc2 = _make_sc_call()(x, new_weight, orig_weight)
    sc1 = sc1.reshape(_NW, _B, _RPW).transpose(1, 0, 2).reshape(_B, _S)
    sc2 = sc2.reshape(_NW, _B, _RPW).transpose(1, 0, 2).reshape(_B, _S)
    ntc = n - _S
    off = _S // _BN
    out_shape = jax.ShapeDtypeStruct((b, ntc), jnp.float32)
    tc1, tc2 = pl.pallas_call(
        _tc_body,
        grid=(ntc // _BN,),
        in_specs=[
            pl.BlockSpec((b, k), lambda j: (0, 0)),
            pl.BlockSpec((_BN, k), lambda j: (j + off, 0)),
            pl.BlockSpec((_BN, k), lambda j: (j + off, 0)),
        ],
        out_specs=[
            pl.BlockSpec((b, _BN), lambda j: (0, j)),
            pl.BlockSpec((b, _BN), lambda j: (0, j)),
        ],
        out_shape=[out_shape, out_shape],
        compiler_params=pltpu.CompilerParams(
            dimension_semantics=("arbitrary",)),
    )(x, new_weight, orig_weight)
    layer_out = jnp.concatenate([sc1, tc1], axis=1)
    original_layer_output = jnp.concatenate([sc2, tc2], axis=1)
    return (layer_out, original_layer_output)


# speedup vs baseline: 2.4267x; 2.4267x over previous
import jax
import jax.numpy as jnp
from jax.experimental import pallas as pl
from jax.experimental.pallas import tpu as pltpu

_BN = 256


def _acke_body(x_ref, nw_ref, ow_ref, o1_ref, o2_ref):
    x = x_ref[...]
    dims = (((1,), (1,)), ((), ()))
    o1_ref[...] = jax.lax.dot_general(
        x, nw_ref[...], dims, preferred_element_type=jnp.float32)
    o2_ref[...] = jax.lax.dot_general(
        x, ow_ref[...], dims, preferred_element_type=jnp.float32)


@jax.jit
def kernel(x, new_weight, orig_weight):
    b, k = x.shape
    n = new_weight.shape[0]
    grid = (n // _BN,)
    out_shape = jax.ShapeDtypeStruct((b, n), jnp.float32)
    call = pl.pallas_call(
        _acke_body,
        grid=grid,
        in_specs=[
            pl.BlockSpec((b, k), lambda j: (0, 0)),
            pl.BlockSpec((_BN, k), lambda j: (j, 0)),
            pl.BlockSpec((_BN, k), lambda j: (j, 0)),
        ],
        out_specs=[
            pl.BlockSpec((b, _BN), lambda j: (0, j)),
            pl.BlockSpec((b, _BN), lambda j: (0, j)),
        ],
        out_shape=[out_shape, out_shape],
        compiler_params=pltpu.CompilerParams(
            dimension_semantics=("arbitrary",)),
    )
    layer_out, original_layer_output = call(x, new_weight, orig_weight)
    return (layer_out, original_layer_output)
